# Initial kernel scaffold; baseline (speedup 1.0000x reference)
import jax
import jax.numpy as jnp
from jax.experimental import pallas as pl


def kernel(context, targets, table):
    raise NotImplementedError("write your pallas kernel here")


# speedup vs baseline: 1.7511x; 1.7511x over previous
import functools

import jax
import jax.numpy as jnp
from jax import lax
from jax.experimental import pallas as pl
from jax.experimental.pallas import tpu as pltpu
from jax.experimental.pallas import tpu_sc as plsc

V = 1000
N_TOK = 204800
NC, NS, L = 2, 16, 16
NW = NC * NS
NB = N_TOK // NW
CH = 32
NCHUNK = NB // CH


def _lse_body(tab_ref, lse_ref):
    x = tab_ref[...]
    m = jnp.max(x, axis=1, keepdims=True)
    s = jnp.sum(jnp.exp(x - m), axis=1, keepdims=True)
    lse_ref[...] = m + jnp.log(s)


def _row_logsumexp(table):
    return pl.pallas_call(
        _lse_body,
        out_shape=jax.ShapeDtypeStruct((V, 1), jnp.float32),
    )(table).reshape(V)


def _sc_body(table_hbm, ctx_hbm, tgt_hbm, lse_hbm,
             out_hbm, part_hbm,
             ctx_v, tgt_v, lse_v, rows0, rows1, acc,
             in0, in1, out0, out1):
    wid = lax.axis_index("s") * NC + lax.axis_index("c")
    base = wid * NB

    pltpu.sync_copy(ctx_hbm.at[pl.ds(base, NB)], ctx_v)
    pltpu.sync_copy(tgt_hbm.at[pl.ds(base, NB)], tgt_v)
    pltpu.sync_copy(lse_hbm, lse_v)
    acc[...] = jnp.zeros((L,), jnp.float32)

    rows = (rows0, rows1)
    sin = (in0, in1)
    sout = (out0, out1)

    def start_gather(g, p):
        pltpu.make_async_copy(
            table_hbm.at[ctx_v.at[pl.ds(g * CH, CH)]], rows[p], sin[p]
        ).start()

    def wait_gather(p):
        pltpu.make_async_copy(
            table_hbm.at[ctx_v.at[pl.ds(0, CH)]], rows[p], sin[p]
        ).wait()

    def start_out(g, p):
        pltpu.make_async_copy(
            rows[p], out_hbm.at[pl.ds(base + g * CH, CH)], sout[p]
        ).start()

    def wait_out(p):
        pltpu.make_async_copy(
            rows[p], out_hbm.at[pl.ds(base, CH)], sout[p]
        ).wait()

    def loss_chunk(g, p):
        off = g * CH
        for j in range(CH // L):
            ctx16 = ctx_v[pl.ds(off + j * L, L)]
            tgt16 = tgt_v[pl.ds(off + j * L, L)]
            rid = lax.iota(jnp.int32, L) + j * L
            vals = plsc.load_gather(rows[p], [rid, tgt16])
            lsec = plsc.load_gather(lse_v, [ctx16])
            acc[...] = acc[...] + (lsec - vals)

    start_gather(0, 0)
    start_gather(1, 1)
    wait_gather(0)
    loss_chunk(0, 0)
    start_out(0, 0)

    def pair_body(g2, _):
        for k in range(2):
            g = 1 + g2 * 2 + k
            p = (1 + k) % 2
            wait_out(1 - p)
            start_gather(g + 1, 1 - p)
            wait_gather(p)
            loss_chunk(g, p)
            start_out(g, p)
        return 0

    lax.fori_loop(0, (NCHUNK - 2) // 2, pair_body, 0)

    wait_gather(1)
    loss_chunk(NCHUNK - 1, 1)
    start_out(NCHUNK - 1, 1)

    wait_out(0)
    wait_out(1)
    pltpu.sync_copy(acc, part_hbm.at[pl.ds(wid * L, L)])


@functools.partial(jax.jit, static_argnums=())
def kernel(context, targets, table):
    ctx_flat = context.reshape(N_TOK)
    tgt_flat = targets.reshape(N_TOK)
    lse = _row_logsumexp(table)

    mesh = plsc.VectorSubcoreMesh(core_axis_name="c", subcore_axis_name="s")
    sc = pl.kernel(
        _sc_body,
        out_type=(
            jax.ShapeDtypeStruct((N_TOK, V), jnp.float32),
            jax.ShapeDtypeStruct((NW * L,), jnp.float32),
        ),
        mesh=mesh,
        compiler_params=pltpu.CompilerParams(
            needs_layout_passes=False, use_tc_tiling_on_sc=False),
        scratch_types=[
            pltpu.VMEM((NB,), jnp.int32),
            pltpu.VMEM((NB,), jnp.int32),
            pltpu.VMEM((V,), jnp.float32),
            pltpu.VMEM((CH, V), jnp.float32),
            pltpu.VMEM((CH, V), jnp.float32),
            pltpu.VMEM((L,), jnp.float32),
            pltpu.SemaphoreType.DMA,
            pltpu.SemaphoreType.DMA,
            pltpu.SemaphoreType.DMA,
            pltpu.SemaphoreType.DMA,
        ],
    )
    logits2, partials = sc(table, ctx_flat, tgt_flat, lse)
    loss = jnp.sum(partials) / jnp.float32(N_TOK)
    return (logits2, loss)
